# baseline (speedup 1.0000x reference)
import jax
import jax.numpy as jnp
from jax import lax
from jax.experimental import pallas as pl
from jax.experimental.pallas import tpu as pltpu
from jax.experimental.pallas import tpu_sc as plsc

N_NODES = 50000
N_EDGES = 1600000
T_STEPS = 40
DT = 0.02

NC = 2
NS = 16
NW = NC * NS
NPAD = 50176
EPT = N_EDGES // NW
EB = 2000
NB = EPT // EB
NPAIR = NB // 2
NS_SLICE = NPAD // NS
NW_SLICE = NPAD // NW

_mesh = plsc.VectorSubcoreMesh(core_axis_name="c", subcore_axis_name="s")


def _step_body(v_hbm, part_hbm, bias_hbm, al_hbm, xt_hbm,
               src_hbm, tgt_hbm, w_hbm,
               vout_hbm, act_hbm, afull_hbm, pout_hbm,
               acc_sh, a_t,
               src0, tgt0, w0, msg0, src1, tgt1, w1, msg1,
               v_v, p0_v, p1_v, b_v, al_v, x_v, a_v,
               sem_n, sem_a, sem_l0, sem_l1, sem_s):
    c = lax.axis_index("c")
    s = lax.axis_index("s")
    sl = pl.ds(s * NS_SLICE, NS_SLICE)
    base = (c * NS + s) * EPT

    def start_lin(b, sb, tb, wb, sem):
        off = base + b * EB
        pltpu.async_copy(src_hbm.at[pl.ds(off, EB)], sb, sem)
        pltpu.async_copy(tgt_hbm.at[pl.ds(off, EB)], tb, sem)
        pltpu.async_copy(w_hbm.at[pl.ds(off, EB)], wb, sem)

    def wait_lin(sb, tb, wb, sem):
        pltpu.make_async_copy(src_hbm.at[pl.ds(base, EB)], sb, sem).wait()
        pltpu.make_async_copy(tgt_hbm.at[pl.ds(base, EB)], tb, sem).wait()
        pltpu.make_async_copy(w_hbm.at[pl.ds(base, EB)], wb, sem).wait()

    start_lin(0, src0, tgt0, w0, sem_l0)
    start_lin(1, src1, tgt1, w1, sem_l1)

    pltpu.async_copy(v_hbm.at[sl], v_v, sem_n)
    pltpu.async_copy(part_hbm.at[sl], p0_v, sem_n)
    pltpu.async_copy(part_hbm.at[pl.ds(NPAD + s * NS_SLICE, NS_SLICE)], p1_v, sem_n)
    pltpu.async_copy(bias_hbm.at[sl], b_v, sem_n)
    pltpu.async_copy(al_hbm.at[sl], al_v, sem_n)
    pltpu.async_copy(xt_hbm.at[sl], x_v, sem_n)
    pltpu.make_async_copy(v_hbm.at[sl], v_v, sem_n).wait()
    pltpu.make_async_copy(part_hbm.at[sl], p0_v, sem_n).wait()
    pltpu.make_async_copy(part_hbm.at[sl], p1_v, sem_n).wait()
    pltpu.make_async_copy(bias_hbm.at[sl], b_v, sem_n).wait()
    pltpu.make_async_copy(al_hbm.at[sl], al_v, sem_n).wait()
    pltpu.make_async_copy(xt_hbm.at[sl], x_v, sem_n).wait()

    for i in range(NS_SLICE // 16):
        ii = pl.ds(i * 16, 16)
        v = v_v[ii]
        vn = v + al_v[ii] * (p0_v[ii] + p1_v[ii] + b_v[ii] + x_v[ii] - v)
        v_v[ii] = vn
        a_v[ii] = jnp.maximum(vn, 0.0)

    half = pl.ds(c * NW_SLICE, NW_SLICE)
    out_half = pl.ds(s * NS_SLICE + c * NW_SLICE, NW_SLICE)
    pltpu.async_copy(a_v, afull_hbm.at[pl.ds(c * NPAD + s * NS_SLICE, NS_SLICE)], sem_n)
    pltpu.async_copy(v_v.at[half], vout_hbm.at[out_half], sem_n)
    pltpu.async_copy(a_v.at[half], act_hbm.at[out_half], sem_n)
    for i in range(NS_SLICE // 16):
        p0_v[pl.ds(i * 16, 16)] = jnp.zeros((16,), jnp.float32)
    pltpu.sync_copy(p0_v, acc_sh.at[sl])
    pltpu.make_async_copy(a_v, afull_hbm.at[pl.ds(c * NPAD, NS_SLICE)], sem_n).wait()
    pltpu.make_async_copy(v_v.at[half], vout_hbm.at[out_half], sem_n).wait()
    pltpu.make_async_copy(a_v.at[half], act_hbm.at[out_half], sem_n).wait()
    plsc.subcore_barrier()

    cpy_a = pltpu.async_copy(afull_hbm.at[pl.ds(c * NPAD, NPAD)], a_t, sem_a)

    def compute(sb, wb, mb):
        for i in range(EB // 16):
            ii = pl.ds(i * 16, 16)
            g = plsc.load_gather(a_t, [sb[ii]])
            mb[ii] = wb[ii] * g

    cpy_a.wait()

    def body(i, carry):
        wait_lin(src0, tgt0, w0, sem_l0)
        compute(src0, w0, msg0)
        d0 = pltpu.async_copy(msg0, acc_sh.at[tgt0], sem_s, add=True)
        wait_lin(src1, tgt1, w1, sem_l1)
        compute(src1, w1, msg1)
        d0.wait()
        start_lin(2 * i + 2, src0, tgt0, w0, sem_l0)
        d1 = pltpu.async_copy(msg1, acc_sh.at[tgt1], sem_s, add=True)
        d1.wait()
        @pl.when(i < NPAIR - 1)
        def _():
            start_lin(2 * i + 3, src1, tgt1, w1, sem_l1)
        return carry

    lax.fori_loop(0, NPAIR, body, 0)

    wait_lin(src0, tgt0, w0, sem_l0)
    compute(src0, w0, msg0)
    pltpu.sync_copy(msg0, acc_sh.at[tgt0], add=True)

    plsc.subcore_barrier()
    pltpu.sync_copy(acc_sh.at[sl], p0_v)
    pltpu.sync_copy(p0_v, pout_hbm.at[pl.ds(c * NPAD + s * NS_SLICE, NS_SLICE)])


_step_call = pl.kernel(
    _step_body,
    out_type=(jax.ShapeDtypeStruct((NPAD,), jnp.float32),
              jax.ShapeDtypeStruct((NPAD,), jnp.float32),
              jax.ShapeDtypeStruct((NC * NPAD,), jnp.float32),
              jax.ShapeDtypeStruct((NC * NPAD,), jnp.float32)),
    mesh=_mesh,
    scratch_types=[
        pltpu.VMEM_SHARED((NPAD,), jnp.float32),
        pltpu.VMEM((NPAD,), jnp.float32),
        pltpu.VMEM((EB,), jnp.int32),
        pltpu.VMEM((EB,), jnp.int32),
        pltpu.VMEM((EB,), jnp.float32),
        pltpu.VMEM((EB,), jnp.float32),
        pltpu.VMEM((EB,), jnp.int32),
        pltpu.VMEM((EB,), jnp.int32),
        pltpu.VMEM((EB,), jnp.float32),
        pltpu.VMEM((EB,), jnp.float32),
        pltpu.VMEM((NS_SLICE,), jnp.float32),
        pltpu.VMEM((NS_SLICE,), jnp.float32),
        pltpu.VMEM((NS_SLICE,), jnp.float32),
        pltpu.VMEM((NS_SLICE,), jnp.float32),
        pltpu.VMEM((NS_SLICE,), jnp.float32),
        pltpu.VMEM((NS_SLICE,), jnp.float32),
        pltpu.VMEM((NS_SLICE,), jnp.float32),
        pltpu.SemaphoreType.DMA,
        pltpu.SemaphoreType.DMA,
        pltpu.SemaphoreType.DMA,
        pltpu.SemaphoreType.DMA,
        pltpu.SemaphoreType.DMA,
    ],
    name="net_step",
    compiler_params=pltpu.CompilerParams(needs_layout_passes=False),
)


@jax.jit
def _run(x, src, tgt, w, bias_p, alpha_p):
    v0 = jnp.full((NPAD,), 0.5, jnp.float32)
    part0 = jnp.zeros((NC * NPAD,), jnp.float32)
    zeros_p = jnp.zeros((NPAD,), jnp.float32)
    pad = NPAD - N_NODES
    xp = jnp.pad(x, ((0, 0), (0, pad)))

    v, _, _, part = _step_call(v0, part0, bias_p, zeros_p, zeros_p, src, tgt, w)

    def step(carry, x_t):
        v, part = carry
        v, act, _, part = _step_call(v, part, bias_p, alpha_p, x_t, src, tgt, w)
        return (v, part), act

    _, acts = lax.scan(step, (v, part), xp)
    return acts[:, :N_NODES]


def kernel(x, edge_index, bias, time_const, sign, syn_count, syn_strength):
    src = edge_index[0]
    tgt = edge_index[1]
    w = sign * jnp.maximum(syn_count, 0.0) * jnp.maximum(syn_strength, 0.0)
    alpha = DT / jnp.maximum(time_const, 1e-3)
    pad = NPAD - N_NODES
    bias_p = jnp.pad(bias, (0, pad))
    alpha_p = jnp.pad(alpha, (0, pad))
    return _run(x, src, tgt, w, bias_p, alpha_p)


# speedup vs baseline: 1.0454x; 1.0454x over previous
import jax
import jax.numpy as jnp
from jax import lax
from jax.experimental import pallas as pl
from jax.experimental.pallas import tpu as pltpu
from jax.experimental.pallas import tpu_sc as plsc

N_NODES = 50000
N_EDGES = 1600000
T_STEPS = 40
DT = 0.02

NC = 2
NS = 16
NW = NC * NS
NPAD = 50176
EPT = N_EDGES // NW
EB = 2000
NB = EPT // EB
NTRI = NB // 3
NS_SLICE = NPAD // NS
NW_SLICE = NPAD // NW

_mesh = plsc.VectorSubcoreMesh(core_axis_name="c", subcore_axis_name="s")


def _step_body(v_hbm, part_hbm, bias_hbm, al_hbm, xt_hbm,
               src_hbm, tgt_hbm, w_hbm,
               vout_hbm, act_hbm, afull_hbm, pout_hbm,
               acc_sh, a_t,
               src0, tgt0, w0, msg0, src1, tgt1, w1, msg1,
               src2, tgt2, w2, msg2,
               v_v, p0_v, p1_v, b_v, al_v, x_v, a_v,
               sem_n, sem_a, sem_l0, sem_l1, sem_l2,
               sem_s0, sem_s1, sem_s2):
    c = lax.axis_index("c")
    s = lax.axis_index("s")
    sl = pl.ds(s * NS_SLICE, NS_SLICE)
    base = (c * NS + s) * EPT

    def start_lin(b, sb, tb, wb, sem):
        off = base + b * EB
        pltpu.async_copy(src_hbm.at[pl.ds(off, EB)], sb, sem)
        pltpu.async_copy(tgt_hbm.at[pl.ds(off, EB)], tb, sem)
        pltpu.async_copy(w_hbm.at[pl.ds(off, EB)], wb, sem)

    def wait_lin(sb, tb, wb, sem):
        pltpu.make_async_copy(src_hbm.at[pl.ds(base, EB)], sb, sem).wait()
        pltpu.make_async_copy(tgt_hbm.at[pl.ds(base, EB)], tb, sem).wait()
        pltpu.make_async_copy(w_hbm.at[pl.ds(base, EB)], wb, sem).wait()

    start_lin(0, src0, tgt0, w0, sem_l0)
    start_lin(1, src1, tgt1, w1, sem_l1)

    pltpu.async_copy(v_hbm.at[sl], v_v, sem_n)
    pltpu.async_copy(part_hbm.at[sl], p0_v, sem_n)
    pltpu.async_copy(part_hbm.at[pl.ds(NPAD + s * NS_SLICE, NS_SLICE)], p1_v, sem_n)
    pltpu.async_copy(bias_hbm.at[sl], b_v, sem_n)
    pltpu.async_copy(al_hbm.at[sl], al_v, sem_n)
    pltpu.async_copy(xt_hbm.at[sl], x_v, sem_n)
    pltpu.make_async_copy(v_hbm.at[sl], v_v, sem_n).wait()
    pltpu.make_async_copy(part_hbm.at[sl], p0_v, sem_n).wait()
    pltpu.make_async_copy(part_hbm.at[sl], p1_v, sem_n).wait()
    pltpu.make_async_copy(bias_hbm.at[sl], b_v, sem_n).wait()
    pltpu.make_async_copy(al_hbm.at[sl], al_v, sem_n).wait()
    pltpu.make_async_copy(xt_hbm.at[sl], x_v, sem_n).wait()

    for i in range(NS_SLICE // 16):
        ii = pl.ds(i * 16, 16)
        v = v_v[ii]
        vn = v + al_v[ii] * (p0_v[ii] + p1_v[ii] + b_v[ii] + x_v[ii] - v)
        v_v[ii] = vn
        a_v[ii] = jnp.maximum(vn, 0.0)

    half = pl.ds(c * NW_SLICE, NW_SLICE)
    out_half = pl.ds(s * NS_SLICE + c * NW_SLICE, NW_SLICE)
    pltpu.async_copy(a_v, afull_hbm.at[pl.ds(c * NPAD + s * NS_SLICE, NS_SLICE)], sem_n)
    pltpu.async_copy(v_v.at[half], vout_hbm.at[out_half], sem_n)
    pltpu.async_copy(a_v.at[half], act_hbm.at[out_half], sem_n)
    for i in range(NS_SLICE // 16):
        p0_v[pl.ds(i * 16, 16)] = jnp.zeros((16,), jnp.float32)
    pltpu.sync_copy(p0_v, acc_sh.at[sl])
    pltpu.make_async_copy(a_v, afull_hbm.at[pl.ds(c * NPAD, NS_SLICE)], sem_n).wait()
    pltpu.make_async_copy(v_v.at[half], vout_hbm.at[out_half], sem_n).wait()
    pltpu.make_async_copy(a_v.at[half], act_hbm.at[out_half], sem_n).wait()
    plsc.subcore_barrier()

    cpy_a = pltpu.async_copy(afull_hbm.at[pl.ds(c * NPAD, NPAD)], a_t, sem_a)

    def compute(sb, wb, mb):
        for i in range(EB // 16):
            ii = pl.ds(i * 16, 16)
            g = plsc.load_gather(a_t, [sb[ii]])
            mb[ii] = wb[ii] * g

    def issue_sc(mb, tb, sem):
        return pltpu.async_copy(mb, acc_sh.at[tb], sem, add=True)

    def wait_sc(mb, tb, sem):
        pltpu.make_async_copy(mb, acc_sh.at[tb], sem).wait()

    cpy_a.wait()

    def body(i, carry):
        @pl.when(i > 0)
        def _():
            wait_sc(msg2, tgt2, sem_s2)
        start_lin(3 * i + 2, src2, tgt2, w2, sem_l2)
        wait_lin(src0, tgt0, w0, sem_l0)
        compute(src0, w0, msg0)
        issue_sc(msg0, tgt0, sem_s0)
        wait_lin(src1, tgt1, w1, sem_l1)
        compute(src1, w1, msg1)
        issue_sc(msg1, tgt1, sem_s1)
        wait_sc(msg0, tgt0, sem_s0)
        start_lin(3 * i + 3, src0, tgt0, w0, sem_l0)
        wait_lin(src2, tgt2, w2, sem_l2)
        compute(src2, w2, msg2)
        issue_sc(msg2, tgt2, sem_s2)
        wait_sc(msg1, tgt1, sem_s1)
        @pl.when(i < NTRI - 1)
        def _():
            start_lin(3 * i + 4, src1, tgt1, w1, sem_l1)
        return carry

    lax.fori_loop(0, NTRI, body, 0)

    wait_lin(src0, tgt0, w0, sem_l0)
    compute(src0, w0, msg0)
    wait_sc(msg2, tgt2, sem_s2)
    pltpu.sync_copy(msg0, acc_sh.at[tgt0], add=True)

    plsc.subcore_barrier()
    pltpu.sync_copy(acc_sh.at[sl], p0_v)
    pltpu.sync_copy(p0_v, pout_hbm.at[pl.ds(c * NPAD + s * NS_SLICE, NS_SLICE)])


_step_call = pl.kernel(
    _step_body,
    out_type=(jax.ShapeDtypeStruct((NPAD,), jnp.float32),
              jax.ShapeDtypeStruct((NPAD,), jnp.float32),
              jax.ShapeDtypeStruct((NC * NPAD,), jnp.float32),
              jax.ShapeDtypeStruct((NC * NPAD,), jnp.float32)),
    mesh=_mesh,
    scratch_types=[
        pltpu.VMEM_SHARED((NPAD,), jnp.float32),
        pltpu.VMEM((NPAD,), jnp.float32),
        pltpu.VMEM((EB,), jnp.int32),
        pltpu.VMEM((EB,), jnp.int32),
        pltpu.VMEM((EB,), jnp.float32),
        pltpu.VMEM((EB,), jnp.float32),
        pltpu.VMEM((EB,), jnp.int32),
        pltpu.VMEM((EB,), jnp.int32),
        pltpu.VMEM((EB,), jnp.float32),
        pltpu.VMEM((EB,), jnp.float32),
        pltpu.VMEM((EB,), jnp.int32),
        pltpu.VMEM((EB,), jnp.int32),
        pltpu.VMEM((EB,), jnp.float32),
        pltpu.VMEM((EB,), jnp.float32),
        pltpu.VMEM((NS_SLICE,), jnp.float32),
        pltpu.VMEM((NS_SLICE,), jnp.float32),
        pltpu.VMEM((NS_SLICE,), jnp.float32),
        pltpu.VMEM((NS_SLICE,), jnp.float32),
        pltpu.VMEM((NS_SLICE,), jnp.float32),
        pltpu.VMEM((NS_SLICE,), jnp.float32),
        pltpu.VMEM((NS_SLICE,), jnp.float32),
        pltpu.SemaphoreType.DMA,
        pltpu.SemaphoreType.DMA,
        pltpu.SemaphoreType.DMA,
        pltpu.SemaphoreType.DMA,
        pltpu.SemaphoreType.DMA,
        pltpu.SemaphoreType.DMA,
        pltpu.SemaphoreType.DMA,
        pltpu.SemaphoreType.DMA,
    ],
    name="net_step",
    compiler_params=pltpu.CompilerParams(needs_layout_passes=False),
)


@jax.jit
def _run(x, src, tgt, w, bias_p, alpha_p):
    v0 = jnp.full((NPAD,), 0.5, jnp.float32)
    part0 = jnp.zeros((NC * NPAD,), jnp.float32)
    zeros_p = jnp.zeros((NPAD,), jnp.float32)
    pad = NPAD - N_NODES
    xp = jnp.pad(x, ((0, 0), (0, pad)))

    v, _, _, part = _step_call(v0, part0, bias_p, zeros_p, zeros_p, src, tgt, w)

    def step(carry, x_t):
        v, part = carry
        v, act, _, part = _step_call(v, part, bias_p, alpha_p, x_t, src, tgt, w)
        return (v, part), act

    _, acts = lax.scan(step, (v, part), xp)
    return acts[:, :N_NODES]


def kernel(x, edge_index, bias, time_const, sign, syn_count, syn_strength):
    src = edge_index[0]
    tgt = edge_index[1]
    w = sign * jnp.maximum(syn_count, 0.0) * jnp.maximum(syn_strength, 0.0)
    alpha = DT / jnp.maximum(time_const, 1e-3)
    pad = NPAD - N_NODES
    bias_p = jnp.pad(bias, (0, pad))
    alpha_p = jnp.pad(alpha, (0, pad))
    return _run(x, src, tgt, w, bias_p, alpha_p)


# speedup vs baseline: 1.1636x; 1.1130x over previous
import jax
import jax.numpy as jnp
from jax import lax
from jax.experimental import pallas as pl
from jax.experimental.pallas import tpu as pltpu
from jax.experimental.pallas import tpu_sc as plsc

N_NODES = 50000
N_EDGES = 1600000
T_STEPS = 40
DT = 0.02

NC = 2
NS = 16
NW = NC * NS
NPAD = 50176
EPT = N_EDGES // NW
EB = 2000
NB = EPT // EB
NPAIR = NB // 2
NS_SLICE = NPAD // NS
NW_SLICE = NPAD // NW

_mesh = plsc.VectorSubcoreMesh(core_axis_name="c", subcore_axis_name="s")


def _step_body(v_hbm, part_hbm, bias_hbm, al_hbm, xt_hbm,
               src_hbm, tgt_hbm, w_hbm,
               vout_hbm, act_hbm, afull_hbm, pout_hbm,
               acc_sh, a_t,
               src0, tgt0, w0, msg0, src1, tgt1, w1, msg1,
               v_v, p0_v, p1_v, b_v, al_v, x_v, a_v,
               sem_n, sem_a, sem_l0, sem_l1, sem_s):
    c = lax.axis_index("c")
    s = lax.axis_index("s")
    sl = pl.ds(s * NS_SLICE, NS_SLICE)
    base = (c * NS + s) * EPT

    def start_lin(b, sb, tb, wb, sem):
        off = base + b * EB
        pltpu.async_copy(src_hbm.at[pl.ds(off, EB)], sb, sem)
        pltpu.async_copy(tgt_hbm.at[pl.ds(off, EB)], tb, sem)
        pltpu.async_copy(w_hbm.at[pl.ds(off, EB)], wb, sem)

    def wait_lin(sb, tb, wb, sem):
        pltpu.make_async_copy(src_hbm.at[pl.ds(base, EB)], sb, sem).wait()
        pltpu.make_async_copy(tgt_hbm.at[pl.ds(base, EB)], tb, sem).wait()
        pltpu.make_async_copy(w_hbm.at[pl.ds(base, EB)], wb, sem).wait()

    start_lin(0, src0, tgt0, w0, sem_l0)
    start_lin(1, src1, tgt1, w1, sem_l1)

    pltpu.async_copy(v_hbm.at[sl], v_v, sem_n)
    pltpu.async_copy(part_hbm.at[sl], p0_v, sem_n)
    pltpu.async_copy(part_hbm.at[pl.ds(NPAD + s * NS_SLICE, NS_SLICE)], p1_v, sem_n)
    pltpu.async_copy(bias_hbm.at[sl], b_v, sem_n)
    pltpu.async_copy(al_hbm.at[sl], al_v, sem_n)
    pltpu.async_copy(xt_hbm.at[sl], x_v, sem_n)
    pltpu.make_async_copy(v_hbm.at[sl], v_v, sem_n).wait()
    pltpu.make_async_copy(part_hbm.at[sl], p0_v, sem_n).wait()
    pltpu.make_async_copy(part_hbm.at[sl], p1_v, sem_n).wait()
    pltpu.make_async_copy(bias_hbm.at[sl], b_v, sem_n).wait()
    pltpu.make_async_copy(al_hbm.at[sl], al_v, sem_n).wait()
    pltpu.make_async_copy(xt_hbm.at[sl], x_v, sem_n).wait()

    for i in range(NS_SLICE // 16):
        ii = pl.ds(i * 16, 16)
        v = v_v[ii]
        vn = v + al_v[ii] * (p0_v[ii] + p1_v[ii] + b_v[ii] + x_v[ii] - v)
        v_v[ii] = vn
        a_v[ii] = jnp.maximum(vn, 0.0)

    half = pl.ds(c * NW_SLICE, NW_SLICE)
    out_half = pl.ds(s * NS_SLICE + c * NW_SLICE, NW_SLICE)
    pltpu.async_copy(a_v, afull_hbm.at[pl.ds(c * NPAD + s * NS_SLICE, NS_SLICE)], sem_n)
    pltpu.async_copy(v_v.at[half], vout_hbm.at[out_half], sem_n)
    pltpu.async_copy(a_v.at[half], act_hbm.at[out_half], sem_n)
    for i in range(NS_SLICE // 16):
        p0_v[pl.ds(i * 16, 16)] = jnp.zeros((16,), jnp.float32)
    pltpu.sync_copy(p0_v, acc_sh.at[sl])
    pltpu.make_async_copy(a_v, afull_hbm.at[pl.ds(c * NPAD, NS_SLICE)], sem_n).wait()
    pltpu.make_async_copy(v_v.at[half], vout_hbm.at[out_half], sem_n).wait()
    pltpu.make_async_copy(a_v.at[half], act_hbm.at[out_half], sem_n).wait()
    plsc.subcore_barrier()

    cpy_a = pltpu.async_copy(afull_hbm.at[pl.ds(c * NPAD, NPAD)], a_t, sem_a)

    def compute(sb, wb, mb):
        for i in range(EB // 16):
            ii = pl.ds(i * 16, 16)
            g = plsc.load_gather(a_t, [sb[ii]])
            mb[ii] = wb[ii] * g

    cpy_a.wait()

    def body(i, carry):
        wait_lin(src0, tgt0, w0, sem_l0)
        compute(src0, w0, msg0)
        d0 = pltpu.async_copy(msg0, acc_sh.at[tgt0], sem_s, add=True)
        wait_lin(src1, tgt1, w1, sem_l1)
        compute(src1, w1, msg1)
        d0.wait()
        start_lin(2 * i + 2, src0, tgt0, w0, sem_l0)
        d1 = pltpu.async_copy(msg1, acc_sh.at[tgt1], sem_s, add=True)
        d1.wait()
        @pl.when(i < NPAIR - 1)
        def _():
            start_lin(2 * i + 3, src1, tgt1, w1, sem_l1)
        return carry

    lax.fori_loop(0, NPAIR, body, 0)

    wait_lin(src0, tgt0, w0, sem_l0)
    compute(src0, w0, msg0)
    pltpu.sync_copy(msg0, acc_sh.at[tgt0], add=True)

    plsc.subcore_barrier()
    pltpu.sync_copy(acc_sh.at[sl], p0_v)
    pltpu.sync_copy(p0_v, pout_hbm.at[pl.ds(c * NPAD + s * NS_SLICE, NS_SLICE)])


_step_call = pl.kernel(
    _step_body,
    out_type=(jax.ShapeDtypeStruct((NPAD,), jnp.float32),
              jax.ShapeDtypeStruct((NPAD,), jnp.float32),
              jax.ShapeDtypeStruct((NC * NPAD,), jnp.float32),
              jax.ShapeDtypeStruct((NC * NPAD,), jnp.float32)),
    mesh=_mesh,
    scratch_types=[
        pltpu.VMEM_SHARED((NPAD,), jnp.float32),
        pltpu.VMEM((NPAD,), jnp.float32),
        pltpu.VMEM((EB,), jnp.int32),
        pltpu.VMEM((EB,), jnp.int32),
        pltpu.VMEM((EB,), jnp.float32),
        pltpu.VMEM((EB,), jnp.float32),
        pltpu.VMEM((EB,), jnp.int32),
        pltpu.VMEM((EB,), jnp.int32),
        pltpu.VMEM((EB,), jnp.float32),
        pltpu.VMEM((EB,), jnp.float32),
        pltpu.VMEM((NS_SLICE,), jnp.float32),
        pltpu.VMEM((NS_SLICE,), jnp.float32),
        pltpu.VMEM((NS_SLICE,), jnp.float32),
        pltpu.VMEM((NS_SLICE,), jnp.float32),
        pltpu.VMEM((NS_SLICE,), jnp.float32),
        pltpu.VMEM((NS_SLICE,), jnp.float32),
        pltpu.VMEM((NS_SLICE,), jnp.float32),
        pltpu.SemaphoreType.DMA,
        pltpu.SemaphoreType.DMA,
        pltpu.SemaphoreType.DMA,
        pltpu.SemaphoreType.DMA,
        pltpu.SemaphoreType.DMA,
    ],
    name="net_step",
    compiler_params=pltpu.CompilerParams(needs_layout_passes=False),
)


def _node_only_body(v_hbm, part_hbm, bias_hbm, al_hbm, xt_hbm,
                    act_hbm, v_v, p0_v, p1_v, b_v, al_v, x_v, sem_n):
    c = lax.axis_index("c")
    s = lax.axis_index("s")
    sl = pl.ds(s * NS_SLICE, NS_SLICE)
    pltpu.async_copy(v_hbm.at[sl], v_v, sem_n)
    pltpu.async_copy(part_hbm.at[sl], p0_v, sem_n)
    pltpu.async_copy(part_hbm.at[pl.ds(NPAD + s * NS_SLICE, NS_SLICE)], p1_v, sem_n)
    pltpu.async_copy(bias_hbm.at[sl], b_v, sem_n)
    pltpu.async_copy(al_hbm.at[sl], al_v, sem_n)
    pltpu.async_copy(xt_hbm.at[sl], x_v, sem_n)
    for _ in range(6):
        pltpu.make_async_copy(v_hbm.at[sl], v_v, sem_n).wait()
    for i in range(NS_SLICE // 16):
        ii = pl.ds(i * 16, 16)
        v = v_v[ii]
        vn = v + al_v[ii] * (p0_v[ii] + p1_v[ii] + b_v[ii] + x_v[ii] - v)
        v_v[ii] = jnp.maximum(vn, 0.0)
    half = pl.ds(c * NW_SLICE, NW_SLICE)
    out_half = pl.ds(s * NS_SLICE + c * NW_SLICE, NW_SLICE)
    pltpu.sync_copy(v_v.at[half], act_hbm.at[out_half])


_node_only_call = pl.kernel(
    _node_only_body,
    out_type=jax.ShapeDtypeStruct((NPAD,), jnp.float32),
    mesh=_mesh,
    scratch_types=[
        pltpu.VMEM((NS_SLICE,), jnp.float32),
        pltpu.VMEM((NS_SLICE,), jnp.float32),
        pltpu.VMEM((NS_SLICE,), jnp.float32),
        pltpu.VMEM((NS_SLICE,), jnp.float32),
        pltpu.VMEM((NS_SLICE,), jnp.float32),
        pltpu.VMEM((NS_SLICE,), jnp.float32),
        pltpu.SemaphoreType.DMA,
    ],
    name="net_node_final",
    compiler_params=pltpu.CompilerParams(needs_layout_passes=False),
)


@jax.jit
def _run(x, src, tgt, w, bias_p, alpha_p):
    v0 = jnp.full((NPAD,), 0.5, jnp.float32)
    part0 = jnp.zeros((NC * NPAD,), jnp.float32)
    zeros_p = jnp.zeros((NPAD,), jnp.float32)
    pad = NPAD - N_NODES
    xp = jnp.pad(x, ((0, 0), (0, pad)))

    v, _, _, part = _step_call(v0, part0, bias_p, zeros_p, zeros_p, src, tgt, w)

    acts = []
    for t in range(T_STEPS - 1):
        v, act, _, part = _step_call(v, part, bias_p, alpha_p, xp[t], src, tgt, w)
        acts.append(act)
    acts.append(_node_only_call(v, part, bias_p, alpha_p, xp[T_STEPS - 1]))
    return jnp.stack(acts)[:, :N_NODES]


def kernel(x, edge_index, bias, time_const, sign, syn_count, syn_strength):
    src = edge_index[0]
    tgt = edge_index[1]
    w = sign * jnp.maximum(syn_count, 0.0) * jnp.maximum(syn_strength, 0.0)
    alpha = DT / jnp.maximum(time_const, 1e-3)
    pad = NPAD - N_NODES
    bias_p = jnp.pad(bias, (0, pad))
    alpha_p = jnp.pad(alpha, (0, pad))
    return _run(x, src, tgt, w, bias_p, alpha_p)
